# Initial kernel scaffold; baseline (speedup 1.0000x reference)
import jax
import jax.numpy as jnp
from jax.experimental import pallas as pl


def kernel(im0, grid):
    raise NotImplementedError("write your pallas kernel here")


# speedup vs baseline: 1.6890x; 1.6890x over previous
import functools

import jax
import jax.numpy as jnp
from jax import lax
from jax.experimental import pallas as pl
from jax.experimental.pallas import tpu as pltpu
from jax.experimental.pallas import tpu_sc as plsc

B, C, H, W = 4, 96, 224, 224
CP = 128
HW = H * W
PAD = 256
HWP = HW + PAD
P = B * HW
NC, NS = 2, 16
NW = NC * NS
PPW = P // NW
CHUNK = 128
NCHUNK = PPW // CHUNK
NG = CHUNK // 16


def _sc_body(table, gx, gy, out,
             gx_v, gy_v, i00, i01, i10, i11, wx_v, wy_v,
             r00, r01, r10, r11, out_v, sem):
    core = lax.axis_index("c")
    sub = lax.axis_index("s")
    wid = sub * NC + core
    base_pt = wid * PPW
    row_base = (wid // (HW // PPW)) * HWP
    iota16 = lax.iota(jnp.int32, 16)

    def chunk_body(t, carry):
        start = base_pt + t * CHUNK
        pltpu.sync_copy(gx.at[pl.ds(start, CHUNK)], gx_v)
        pltpu.sync_copy(gy.at[pl.ds(start, CHUNK)], gy_v)

        for g in range(NG):
            sl = pl.ds(g * 16, 16)
            xv = gx_v[sl] * jnp.float32(W - 1)
            yv = gy_v[sl] * jnp.float32(H - 1)
            x0 = xv.astype(jnp.int32)
            y0 = yv.astype(jnp.int32)
            wx_v[sl] = xv - x0.astype(jnp.float32)
            wy_v[sl] = yv - y0.astype(jnp.float32)
            idx = row_base + y0 * W + x0
            i00[sl] = idx
            i01[sl] = idx + 1
            i10[sl] = idx + W
            i11[sl] = idx + (W + 1)

        d0 = pltpu.async_copy(table.at[i00], r00, sem)
        d1 = pltpu.async_copy(table.at[i01], r01, sem)
        d2 = pltpu.async_copy(table.at[i10], r10, sem)
        d3 = pltpu.async_copy(table.at[i11], r11, sem)
        d0.wait()
        d1.wait()
        d2.wait()
        d3.wait()

        def grp_combine(g, carry2):
            sl = pl.ds(g * 16, 16)
            wx = wx_v[sl]
            wy = wy_v[sl]
            one_m_wx = 1.0 - wx
            one_m_wy = 1.0 - wy
            w00 = one_m_wx * one_m_wy
            w01 = wx * one_m_wy
            w10 = one_m_wx * wy
            w11 = wx * wy
            rowi = g * 16 + iota16
            cv = jnp.zeros((16,), jnp.int32)
            for c in range(C):
                v = (w00 * plsc.load_gather(r00, [rowi, cv])
                     + w01 * plsc.load_gather(r01, [rowi, cv])
                     + w10 * plsc.load_gather(r10, [rowi, cv])
                     + w11 * plsc.load_gather(r11, [rowi, cv]))
                plsc.store_scatter(out_v, [rowi, cv], v)
                cv = cv + 1
            return carry2

        lax.fori_loop(0, NG, grp_combine, 0)
        pltpu.sync_copy(out_v, out.at[pl.ds(start, CHUNK)])
        return carry

    lax.fori_loop(0, NCHUNK, chunk_body, 0)


_MESH = plsc.VectorSubcoreMesh(core_axis_name="c", subcore_axis_name="s",
                               num_cores=NC, num_subcores=NS)

_sc_interp = pl.kernel(
    _sc_body,
    out_type=jax.ShapeDtypeStruct((P, C), jnp.float32),
    mesh=_MESH,
    compiler_params=pltpu.CompilerParams(needs_layout_passes=False),
    scratch_types=[
        pltpu.VMEM((CHUNK,), jnp.float32),
        pltpu.VMEM((CHUNK,), jnp.float32),
        pltpu.VMEM((CHUNK,), jnp.int32),
        pltpu.VMEM((CHUNK,), jnp.int32),
        pltpu.VMEM((CHUNK,), jnp.int32),
        pltpu.VMEM((CHUNK,), jnp.int32),
        pltpu.VMEM((CHUNK,), jnp.float32),
        pltpu.VMEM((CHUNK,), jnp.float32),
        pltpu.VMEM((CHUNK, CP), jnp.float32),
        pltpu.VMEM((CHUNK, CP), jnp.float32),
        pltpu.VMEM((CHUNK, CP), jnp.float32),
        pltpu.VMEM((CHUNK, CP), jnp.float32),
        pltpu.VMEM((CHUNK, C), jnp.float32),
        pltpu.SemaphoreType.DMA,
    ],
)


@jax.jit
def kernel(im0, grid):
    imt = jnp.transpose(im0, (0, 2, 3, 1)).reshape(B, HW, C)
    imt = jnp.pad(imt, ((0, 0), (0, PAD), (0, CP - C))).reshape(B * HWP, CP)
    g2 = grid.reshape(B, 2, HW)
    gx = g2[:, 0].reshape(P)
    gy = g2[:, 1].reshape(P)
    outf = _sc_interp(imt, gx, gy)
    return outf.reshape(B, H, W, C).transpose(0, 3, 1, 2)


# speedup vs baseline: 2.0296x; 1.2017x over previous
import jax
import jax.numpy as jnp
from jax import lax
from jax.experimental import pallas as pl
from jax.experimental.pallas import tpu as pltpu
from jax.experimental.pallas import tpu_sc as plsc

B, C, H, W = 4, 96, 224, 224
CP = 128
HW = H * W
PAD = 256
HWP = HW + PAD
P = B * HW
NC, NS = 2, 16
NW = NC * NS
PPW = P // NW
WPB = HW // PPW
CHUNK = 64
NCHUNK = PPW // CHUNK
NPAIR = NCHUNK // 2
OROWS = B * C * (HW // CP)
ORPB = HW // CP


def _sc_body(table, gx, gy, out,
             gxall, gyall,
             ia0, ia1, ia2, ia3, wxa, wya, ra0, ra1, ra2, ra3,
             ib0, ib1, ib2, ib3, wxb, wyb, rb0, rb1, rb2, rb3,
             out_t, idx_o, sem_a, sem_b, sem_o):
    core = lax.axis_index("c")
    sub = lax.axis_index("s")
    wid = sub * NC + core
    b = wid // WPB
    row_base = b * HWP
    obase = b * C * ORPB + (wid % WPB) * (PPW // CP)
    iota16 = lax.iota(jnp.int32, 16)

    pltpu.sync_copy(gx.at[pl.ds(wid * PPW, PPW)], gxall)
    pltpu.sync_copy(gy.at[pl.ds(wid * PPW, PPW)], gyall)

    def prep(t, i0, i1, i2, i3, wxs, wys):
        for g in range(CHUNK // 16):
            src = pl.ds(t * CHUNK + g * 16, 16)
            dst = pl.ds(g * 16, 16)
            xv = gxall[src] * jnp.float32(W - 1)
            yv = gyall[src] * jnp.float32(H - 1)
            x0 = xv.astype(jnp.int32)
            y0 = yv.astype(jnp.int32)
            wxs[dst] = xv - x0.astype(jnp.float32)
            wys[dst] = yv - y0.astype(jnp.float32)
            idx = row_base + y0 * W + x0
            i0[dst] = idx
            i1[dst] = idx + 1
            i2[dst] = idx + W
            i3[dst] = idx + (W + 1)

    def issue(i0, i1, i2, i3, r0, r1, r2, r3, sem):
        pltpu.async_copy(table.at[i0], r0, sem)
        pltpu.async_copy(table.at[i1], r1, sem)
        pltpu.async_copy(table.at[i2], r2, sem)
        pltpu.async_copy(table.at[i3], r3, sem)

    def drain_gather(r0, r1, r2, r3, sem):
        for r in (r0, r1, r2, r3):
            pltpu.make_async_copy(table.at[pl.ds(0, CHUNK)], r, sem).wait()

    def combine(half, wxs, wys, r0, r1, r2, r3):
        def grp(g, carry):
            sl = pl.ds(g * 16, 16)
            wx = wxs[sl]
            wy = wys[sl]
            one_m_wx = 1.0 - wx
            one_m_wy = 1.0 - wy
            w00 = one_m_wx * one_m_wy
            w01 = wx * one_m_wy
            w10 = one_m_wx * wy
            w11 = wx * wy
            rowi = g * 16 + iota16
            col = pl.ds(half * CHUNK + g * 16, 16)
            for c in range(C):
                cv = jnp.full((16,), c, jnp.int32)
                v = (w00 * plsc.load_gather(r0, [rowi, cv])
                     + w01 * plsc.load_gather(r1, [rowi, cv])
                     + w10 * plsc.load_gather(r2, [rowi, cv])
                     + w11 * plsc.load_gather(r3, [rowi, cv]))
                out_t[c, col] = v
            return carry

        lax.fori_loop(0, CHUNK // 16, grp, 0)

    def scatter_out(pair):
        blk = obase + pair
        for g in range(C // 16):
            c16 = g * 16 + iota16
            idx_o[pl.ds(g * 16, 16)] = blk + c16 * ORPB
        pltpu.async_copy(out_t, out.at[idx_o], sem_o)

    def drain_out():
        pltpu.make_async_copy(out.at[pl.ds(0, C)], out_t, sem_o).wait()

    prep(0, ia0, ia1, ia2, ia3, wxa, wya)
    issue(ia0, ia1, ia2, ia3, ra0, ra1, ra2, ra3, sem_a)

    def pair_body(i, carry):
        prep(2 * i + 1, ib0, ib1, ib2, ib3, wxb, wyb)
        issue(ib0, ib1, ib2, ib3, rb0, rb1, rb2, rb3, sem_b)
        drain_gather(ra0, ra1, ra2, ra3, sem_a)

        @pl.when(i > 0)
        def _():
            drain_out()

        combine(0, wxa, wya, ra0, ra1, ra2, ra3)

        @pl.when(i < NPAIR - 1)
        def _():
            prep(2 * i + 2, ia0, ia1, ia2, ia3, wxa, wya)
            issue(ia0, ia1, ia2, ia3, ra0, ra1, ra2, ra3, sem_a)

        drain_gather(rb0, rb1, rb2, rb3, sem_b)
        combine(1, wxb, wyb, rb0, rb1, rb2, rb3)
        scatter_out(i)
        return carry

    lax.fori_loop(0, NPAIR, pair_body, 0)
    drain_out()


_MESH = plsc.VectorSubcoreMesh(core_axis_name="c", subcore_axis_name="s",
                               num_cores=NC, num_subcores=NS)

_IDX = pltpu.VMEM((CHUNK,), jnp.int32)
_WGT = pltpu.VMEM((CHUNK,), jnp.float32)
_ROWS = pltpu.VMEM((CHUNK, CP), jnp.float32)

_sc_interp = pl.kernel(
    _sc_body,
    out_type=jax.ShapeDtypeStruct((OROWS, CP), jnp.float32),
    mesh=_MESH,
    compiler_params=pltpu.CompilerParams(needs_layout_passes=False),
    scratch_types=[
        pltpu.VMEM((PPW,), jnp.float32),
        pltpu.VMEM((PPW,), jnp.float32),
        _IDX, _IDX, _IDX, _IDX, _WGT, _WGT, _ROWS, _ROWS, _ROWS, _ROWS,
        _IDX, _IDX, _IDX, _IDX, _WGT, _WGT, _ROWS, _ROWS, _ROWS, _ROWS,
        pltpu.VMEM((C, CP), jnp.float32),
        pltpu.VMEM((C,), jnp.int32),
        pltpu.SemaphoreType.DMA,
        pltpu.SemaphoreType.DMA,
        pltpu.SemaphoreType.DMA,
    ],
)


@jax.jit
def kernel(im0, grid):
    imt = jnp.transpose(im0, (0, 2, 3, 1)).reshape(B, HW, C)
    imt = jnp.pad(imt, ((0, 0), (0, PAD), (0, CP - C))).reshape(B * HWP, CP)
    g2 = grid.reshape(B, 2, HW)
    gx = g2[:, 0].reshape(P)
    gy = g2[:, 1].reshape(P)
    outf = _sc_interp(imt, gx, gy)
    return outf.reshape(B, C, H, W)


# speedup vs baseline: 2.1989x; 1.0834x over previous
import jax
import jax.numpy as jnp
from jax import lax
from jax.experimental import pallas as pl
from jax.experimental.pallas import tpu as pltpu
from jax.experimental.pallas import tpu_sc as plsc

B, C, H, W = 4, 96, 224, 224
CP = 128
CPI = CP // 2
HW = H * W
PAD = 256
HWP = HW + PAD
P = B * HW
NC, NS = 2, 16
NW = NC * NS
PPW = P // NW
WPB = HW // PPW
CHUNK = 64
NCHUNK = PPW // CHUNK
NPAIR = NCHUNK // 2
OBLK = 128
OROWS = B * C * (HW // OBLK)
ORPB = HW // OBLK


def _sc_body(table, gx, gy, out,
             gxall, gyall,
             ia0, ia1, w00a, w01a, w10a, w11a, ra0, ra1,
             ib0, ib1, w00b, w01b, w10b, w11b, rb0, rb1,
             out_t, idx_o, sem_a, sem_b, sem_o):
    core = lax.axis_index("c")
    sub = lax.axis_index("s")
    wid = sub * NC + core
    b = wid // WPB
    row_base = b * HWP
    obase = b * C * ORPB + (wid % WPB) * (PPW // OBLK)
    iota16 = lax.iota(jnp.int32, 16)
    zeros16 = jnp.zeros((16,), jnp.int32)
    ch_ev = [32 * k + 2 * iota16 for k in range(C // 32)]
    ch_od = [32 * k + 2 * iota16 + 1 for k in range(C // 32)]

    pltpu.sync_copy(gx.at[pl.ds(wid * PPW, PPW)], gxall)
    pltpu.sync_copy(gy.at[pl.ds(wid * PPW, PPW)], gyall)

    def prep(t, i0, i1, w00s, w01s, w10s, w11s):
        for g in range(CHUNK // 16):
            src = pl.ds(t * CHUNK + g * 16, 16)
            dst = pl.ds(g * 16, 16)
            xv = gxall[src] * jnp.float32(W - 1)
            yv = gyall[src] * jnp.float32(H - 1)
            x0 = xv.astype(jnp.int32)
            y0 = yv.astype(jnp.int32)
            wx = xv - x0.astype(jnp.float32)
            wy = yv - y0.astype(jnp.float32)
            one_m_wx = 1.0 - wx
            one_m_wy = 1.0 - wy
            w00s[dst] = one_m_wx * one_m_wy
            w01s[dst] = wx * one_m_wy
            w10s[dst] = one_m_wx * wy
            w11s[dst] = wx * wy
            idx = row_base + y0 * W + x0
            i0[dst] = idx
            i1[dst] = idx + 1

    def issue(i0, i1, r0, r1, sem):
        pltpu.async_copy(table.at[i0], r0, sem)
        pltpu.async_copy(table.at[i1], r1, sem)

    def drain_gather(r0, r1, sem):
        for r in (r0, r1):
            pltpu.make_async_copy(table.at[pl.ds(0, CHUNK)], r, sem).wait()

    def combine(half, w00s, w01s, w10s, w11s, r0, r1):
        def point(p, carry):
            pw = p + zeros16
            po = (half * CHUNK + p) + zeros16
            b00 = plsc.load_gather(w00s, [pw])
            b01 = plsc.load_gather(w01s, [pw])
            b10 = plsc.load_gather(w10s, [pw])
            b11 = plsc.load_gather(w11s, [pw])
            for k in range(C // 32):
                sl0 = pl.ds(16 * k, 16)
                sl1 = pl.ds(CPI + 16 * k, 16)

                def dec(v):
                    return plsc.unpack(plsc.bitcast(v, jnp.bfloat16),
                                       format=plsc.PackFormat.INTERLEAVED)

                e0, o0 = dec(r0[p, sl0])
                e1, o1 = dec(r1[p, sl0])
                e2, o2 = dec(r0[p, sl1])
                e3, o3 = dec(r1[p, sl1])
                ve = b00 * e0 + b01 * e1 + b10 * e2 + b11 * e3
                vo = b00 * o0 + b01 * o1 + b10 * o2 + b11 * o3
                plsc.store_scatter(out_t, [ch_ev[k], po], ve)
                plsc.store_scatter(out_t, [ch_od[k], po], vo)
            return carry

        lax.fori_loop(0, CHUNK, point, 0)

    def scatter_out(pair):
        blk = obase + pair
        for g in range(C // 16):
            c16 = g * 16 + iota16
            idx_o[pl.ds(g * 16, 16)] = blk + c16 * ORPB
        pltpu.async_copy(out_t, out.at[idx_o], sem_o)

    def drain_out():
        pltpu.make_async_copy(out.at[pl.ds(0, C)], out_t, sem_o).wait()

    prep(0, ia0, ia1, w00a, w01a, w10a, w11a)
    issue(ia0, ia1, ra0, ra1, sem_a)

    def pair_body(i, carry):
        prep(2 * i + 1, ib0, ib1, w00b, w01b, w10b, w11b)
        issue(ib0, ib1, rb0, rb1, sem_b)
        drain_gather(ra0, ra1, sem_a)

        @pl.when(i > 0)
        def _():
            drain_out()

        combine(0, w00a, w01a, w10a, w11a, ra0, ra1)

        @pl.when(i < NPAIR - 1)
        def _():
            prep(2 * i + 2, ia0, ia1, w00a, w01a, w10a, w11a)
            issue(ia0, ia1, ra0, ra1, sem_a)

        drain_gather(rb0, rb1, sem_b)
        combine(1, w00b, w01b, w10b, w11b, rb0, rb1)
        scatter_out(i)
        return carry

    lax.fori_loop(0, NPAIR, pair_body, 0)
    drain_out()


_MESH = plsc.VectorSubcoreMesh(core_axis_name="c", subcore_axis_name="s",
                               num_cores=NC, num_subcores=NS)

_IDX = pltpu.VMEM((CHUNK,), jnp.int32)
_WGT = pltpu.VMEM((CHUNK,), jnp.float32)
_ROWS = pltpu.VMEM((CHUNK, 2 * CPI), jnp.int32)

_sc_interp = pl.kernel(
    _sc_body,
    out_type=jax.ShapeDtypeStruct((OROWS, OBLK), jnp.float32),
    mesh=_MESH,
    compiler_params=pltpu.CompilerParams(needs_layout_passes=False),
    scratch_types=[
        pltpu.VMEM((PPW,), jnp.float32),
        pltpu.VMEM((PPW,), jnp.float32),
        _IDX, _IDX, _WGT, _WGT, _WGT, _WGT, _ROWS, _ROWS,
        _IDX, _IDX, _WGT, _WGT, _WGT, _WGT, _ROWS, _ROWS,
        pltpu.VMEM((C, OBLK), jnp.float32),
        pltpu.VMEM((C,), jnp.int32),
        pltpu.SemaphoreType.DMA,
        pltpu.SemaphoreType.DMA,
        pltpu.SemaphoreType.DMA,
    ],
)


@jax.jit
def kernel(im0, grid):
    a = jnp.transpose(im0.astype(jnp.bfloat16), (0, 2, 3, 1))
    ay = jnp.concatenate(
        [a[:, 1:], jnp.zeros((B, 1, W, C), jnp.bfloat16)], axis=1)
    imt = jnp.stack([a, ay], axis=3).reshape(B, HW, 2, C)
    imt = jnp.pad(imt, ((0, 0), (0, PAD), (0, 0), (0, CP - C)))
    imt = jax.lax.bitcast_convert_type(
        imt.reshape(B * HWP, 2 * CPI, 2), jnp.int32)
    g2 = grid.reshape(B, 2, HW)
    gx = g2[:, 0].reshape(P)
    gy = g2[:, 1].reshape(P)
    outf = _sc_interp(imt, gx, gy)
    return outf.reshape(B, C, H, W)


# speedup vs baseline: 3.8539x; 1.7527x over previous
import jax
import jax.numpy as jnp
from jax import lax
from jax.experimental import pallas as pl
from jax.experimental.pallas import tpu as pltpu
from jax.experimental.pallas import tpu_sc as plsc

B, C, H, W = 4, 96, 224, 224
CP = 128
CPI = CP // 2
HW = H * W
PAD = 256
HWP = HW + PAD
P = B * HW
NC, NS = 2, 16
NW = NC * NS
PPW = P // NW
WPB = HW // PPW
CHUNK = 128
NCHUNK = PPW // CHUNK
NPAIR = (NCHUNK - 1) // 2
OBLK = 128
OROWS = B * C * (HW // OBLK)
ORPB = HW // OBLK
PXR = HWP + W


def _sc_body(table, gxy, out,
             gxall, gyall,
             ia0, ia1, w00a, w01a, w10a, w11a, ra0, ra1,
             ib0, ib1, w00b, w01b, w10b, w11b, rb0, rb1,
             out_t, idx_o, sem_a, sem_b, sem_o):
    core = lax.axis_index("c")
    sub = lax.axis_index("s")
    wid = core * NS + sub
    b = wid // WPB
    row_base = b * HWP
    obase = b * C * ORPB + (wid % WPB) * (PPW // OBLK)
    iota16 = lax.iota(jnp.int32, 16)
    zeros16 = jnp.zeros((16,), jnp.int32)
    ch_ev = [32 * k + 2 * iota16 for k in range(C // 32)]
    ch_od = [32 * k + 2 * iota16 + 1 for k in range(C // 32)]

    inb = (wid % WPB) * PPW
    pltpu.sync_copy(gxy.at[pl.ds(2 * b * HW + inb, PPW)], gxall)
    pltpu.sync_copy(gxy.at[pl.ds((2 * b + 1) * HW + inb, PPW)], gyall)

    def prep(t, i0, i1, w00s, w01s, w10s, w11s):
        for g in range(CHUNK // 16):
            src = pl.ds(t * CHUNK + g * 16, 16)
            dst = pl.ds(g * 16, 16)
            xv = gxall[src] * jnp.float32(W - 1)
            yv = gyall[src] * jnp.float32(H - 1)
            x0 = xv.astype(jnp.int32)
            y0 = yv.astype(jnp.int32)
            wx = xv - x0.astype(jnp.float32)
            wy = yv - y0.astype(jnp.float32)
            one_m_wx = 1.0 - wx
            one_m_wy = 1.0 - wy
            w00s[dst] = one_m_wx * one_m_wy
            w01s[dst] = wx * one_m_wy
            w10s[dst] = one_m_wx * wy
            w11s[dst] = wx * wy
            idx = row_base + y0 * W + x0
            i0[dst] = idx
            i1[dst] = jnp.minimum(idx + 1, row_base + HW - 1)

    def issue(i0, i1, r0, r1, sem):
        pltpu.async_copy(table.at[i0], r0, sem)
        pltpu.async_copy(table.at[i1], r1, sem)

    def drain_gather(r0, r1, sem):
        for r in (r0, r1):
            pltpu.make_async_copy(table.at[pl.ds(0, CHUNK)], r, sem).wait()

    def combine(w00s, w01s, w10s, w11s, r0, r1):
        def point(p, carry):
            pw = p + zeros16
            po = pw
            b00 = plsc.load_gather(w00s, [pw])
            b01 = plsc.load_gather(w01s, [pw])
            b10 = plsc.load_gather(w10s, [pw])
            b11 = plsc.load_gather(w11s, [pw])
            for k in range(C // 32):
                sl0 = pl.ds(16 * k, 16)
                sl1 = pl.ds(CPI + 16 * k, 16)

                def dec(v):
                    return plsc.unpack(plsc.bitcast(v, jnp.bfloat16),
                                       format=plsc.PackFormat.INTERLEAVED)

                e0, o0 = dec(r0[p, sl0])
                e1, o1 = dec(r1[p, sl0])
                e2, o2 = dec(r0[p, sl1])
                e3, o3 = dec(r1[p, sl1])
                ve = b00 * e0 + b01 * e1 + b10 * e2 + b11 * e3
                vo = b00 * o0 + b01 * o1 + b10 * o2 + b11 * o3
                plsc.store_scatter(out_t, [ch_ev[k], po], ve)
                plsc.store_scatter(out_t, [ch_od[k], po], vo)
            return carry

        lax.fori_loop(0, CHUNK, point, 0)

    def scatter_out(pair):
        blk = obase + pair
        for g in range(C // 16):
            c16 = g * 16 + iota16
            idx_o[pl.ds(g * 16, 16)] = blk + c16 * ORPB
        pltpu.async_copy(out_t, out.at[idx_o], sem_o)

    def drain_out():
        pltpu.make_async_copy(out.at[pl.ds(0, C)], out_t, sem_o).wait()

    prep(0, ia0, ia1, w00a, w01a, w10a, w11a)
    issue(ia0, ia1, ra0, ra1, sem_a)

    def pair_body(i, carry):
        prep(2 * i + 1, ib0, ib1, w00b, w01b, w10b, w11b)
        issue(ib0, ib1, rb0, rb1, sem_b)
        drain_gather(ra0, ra1, sem_a)

        @pl.when(i > 0)
        def _():
            drain_out()

        combine(w00a, w01a, w10a, w11a, ra0, ra1)
        scatter_out(2 * i)
        prep(2 * i + 2, ia0, ia1, w00a, w01a, w10a, w11a)
        issue(ia0, ia1, ra0, ra1, sem_a)
        drain_gather(rb0, rb1, sem_b)
        drain_out()
        combine(w00b, w01b, w10b, w11b, rb0, rb1)
        scatter_out(2 * i + 1)
        return carry

    lax.fori_loop(0, NPAIR, pair_body, 0)
    drain_gather(ra0, ra1, sem_a)
    drain_out()
    combine(w00a, w01a, w10a, w11a, ra0, ra1)
    scatter_out(NCHUNK - 1)
    drain_out()


_MESH = plsc.VectorSubcoreMesh(core_axis_name="c", subcore_axis_name="s",
                               num_cores=NC, num_subcores=NS)

_IDX = pltpu.VMEM((CHUNK,), jnp.int32)
_WGT = pltpu.VMEM((CHUNK,), jnp.float32)
_ROWS = pltpu.VMEM((CHUNK, 2 * CPI), jnp.int32)

_sc_interp = pl.kernel(
    _sc_body,
    out_type=jax.ShapeDtypeStruct((OROWS, OBLK), jnp.float32),
    mesh=_MESH,
    compiler_params=pltpu.CompilerParams(needs_layout_passes=False),
    scratch_types=[
        pltpu.VMEM((PPW,), jnp.float32),
        pltpu.VMEM((PPW,), jnp.float32),
        _IDX, _IDX, _WGT, _WGT, _WGT, _WGT, _ROWS, _ROWS,
        _IDX, _IDX, _WGT, _WGT, _WGT, _WGT, _ROWS, _ROWS,
        pltpu.VMEM((C, OBLK), jnp.float32),
        pltpu.VMEM((C,), jnp.int32),
        pltpu.SemaphoreType.DMA,
        pltpu.SemaphoreType.DMA,
        pltpu.SemaphoreType.DMA,
    ],
)


@jax.jit
def kernel(im0, grid):
    a = jnp.transpose(im0.astype(jnp.bfloat16), (0, 2, 3, 1))
    a = jnp.pad(a.reshape(B, HW, C), ((0, 0), (0, PAD + W), (0, CP - C)))
    px = jax.lax.bitcast_convert_type(
        a.reshape(B, PXR, CPI, 2), jnp.int32)
    imt = jnp.concatenate([px[:, :HWP], px[:, W:]], axis=-1)
    imt = imt.reshape(B * HWP, 2 * CPI)
    outf = _sc_interp(imt, grid.reshape(2 * P))
    return outf.reshape(B, C, H, W)
